# baseline (speedup 1.0000x reference)
import jax
import jax.numpy as jnp
from jax.experimental import pallas as pl
from jax.experimental.pallas import tpu as pltpu

_R = 400
_RS = 1000
_H = 128
_LP = jnp.bfloat16


def _supports_body(x_ref, ws_ref, wf_ref, ss_ref, sf_ref):
    xb = x_ref[...].astype(_LP)
    ss_ref[...] = jnp.dot(xb, ws_ref[...],
                          preferred_element_type=jnp.float32).astype(_LP)
    sf_ref[...] = jnp.dot(xb, wf_ref[...],
                          preferred_element_type=jnp.float32).astype(_LP)


def _l1_body(adj_ref, s_ref, b1_ref, wl_ref, wr_ref, t_ref):
    acc = jnp.dot(adj_ref[...].astype(_LP), s_ref[...],
                  preferred_element_type=jnp.float32)
    hx = jnp.maximum(acc + b1_ref[...], 0.0).astype(_LP)
    t_ref[:, :_H] = jnp.dot(hx[:, :_H], wl_ref[...],
                            preferred_element_type=jnp.float32).astype(_LP)
    t_ref[:, _H:] = jnp.dot(hx[:, _H:], wr_ref[...],
                            preferred_element_type=jnp.float32).astype(_LP)


def _l2s_body(adj_ref, t_ref, b2_ref, e_ref):
    e_ref[...] = jnp.dot(adj_ref[...].astype(_LP), t_ref[...],
                         preferred_element_type=jnp.float32) + b2_ref[...]


def _l2f_body(fadj_ref, tf_ref, es_ref, bf2_ref,
              wa1_ref, ba1_ref, wa2_ref, wm_ref, bm_ref, out_ref):
    ef = jnp.dot(fadj_ref[...].astype(_LP), tf_ref[...],
                 preferred_element_type=jnp.float32) + bf2_ref[...]
    es = es_ref[...]
    emb1 = es[:, :_H]
    emb2 = ef[:, :_H]
    xcom = 0.5 * (es[:, _H:] + ef[:, _H:])
    wa1 = wa1_ref[...]
    ba1 = ba1_ref[...]
    wa2 = wa2_ref[...]

    def att(e):
        t = jnp.tanh(jnp.dot(e.astype(_LP), wa1,
                             preferred_element_type=jnp.float32) + ba1)
        return jnp.sum(t.astype(_LP).astype(jnp.float32)
                       * wa2.astype(jnp.float32), axis=1, keepdims=True)

    w1 = att(emb1)
    w2 = att(emb2)
    w3 = att(xcom)
    m = jnp.maximum(jnp.maximum(w1, w2), w3)
    e1 = jnp.exp(w1 - m)
    e2 = jnp.exp(w2 - m)
    e3 = jnp.exp(w3 - m)
    emb = (e1 * emb1 + e2 * emb2 + e3 * xcom) / (e1 + e2 + e3)
    logits = jnp.dot(emb.astype(_LP), wm_ref[...],
                     preferred_element_type=jnp.float32) + bm_ref[...]
    lmax = jnp.max(logits, axis=1, keepdims=True)
    lse = jnp.log(jnp.sum(jnp.exp(logits - lmax), axis=1, keepdims=True)) + lmax
    out_ref[...] = logits - lse


def kernel(x, sadj, fadj, W1_1, b1_1, W1_2, b1_2, W2_1, b2_1, W2_2, b2_2,
           Wc_1, bc_1, Wc_2, bc_2, Wa1, ba1, Wa2, Wm, bm):
    n, f = x.shape
    h = W1_1.shape[1]
    c = Wm.shape[1]
    lp = _LP

    ws1 = jnp.concatenate([W1_1, Wc_1], axis=1).astype(lp)
    wf1 = jnp.concatenate([W2_1, Wc_1], axis=1).astype(lp)
    bs1 = jnp.concatenate([b1_1, bc_1]).reshape(1, 2 * h)
    bf1 = jnp.concatenate([b2_1, bc_1]).reshape(1, 2 * h)
    bs2 = jnp.concatenate([b1_2, bc_2]).reshape(1, 2 * h)
    bf2 = jnp.concatenate([b2_2, bc_2]).reshape(1, 2 * h)

    full = lambda shape: pl.BlockSpec(shape, lambda i: (0, 0))
    sup_spec = full((n, 2 * h))
    adj_spec = pl.BlockSpec((_R, n), lambda i: (i, 0))
    row_spec = pl.BlockSpec((_R, 2 * h), lambda i: (i, 0))
    seq = pltpu.CompilerParams(dimension_semantics=("arbitrary",))
    w2c = Wc_2.astype(lp)
    grid = (n // _R,)

    ss, sf = pl.pallas_call(
        _supports_body,
        grid=(n // _RS,),
        in_specs=[
            pl.BlockSpec((_RS, f), lambda i: (i, 0)),
            full((f, 2 * h)),
            full((f, 2 * h)),
        ],
        out_specs=[
            pl.BlockSpec((_RS, 2 * h), lambda i: (i, 0)),
            pl.BlockSpec((_RS, 2 * h), lambda i: (i, 0)),
        ],
        out_shape=[jax.ShapeDtypeStruct((n, 2 * h), lp)] * 2,
    )(x, ws1, wf1)

    l1_specs = dict(
        grid=grid,
        in_specs=[adj_spec, sup_spec, full((1, 2 * h)),
                  full((h, h)), full((h, h))],
        out_specs=row_spec,
        out_shape=jax.ShapeDtypeStruct((n, 2 * h), lp),
        compiler_params=seq,
    )
    ts = pl.pallas_call(_l1_body, **l1_specs)(
        sadj, ss, bs1, W1_2.astype(lp), w2c)
    tf = pl.pallas_call(_l1_body, **l1_specs)(
        fadj, sf, bf1, W2_2.astype(lp), w2c)

    es = pl.pallas_call(
        _l2s_body,
        grid=grid,
        in_specs=[adj_spec, sup_spec, full((1, 2 * h))],
        out_specs=row_spec,
        out_shape=jax.ShapeDtypeStruct((n, 2 * h), jnp.float32),
        compiler_params=seq,
    )(sadj, ts, bs2)

    out = pl.pallas_call(
        _l2f_body,
        grid=grid,
        in_specs=[
            adj_spec, sup_spec, row_spec, full((1, 2 * h)),
            full((h, Wa1.shape[1])), full((1, Wa1.shape[1])),
            full((1, Wa2.shape[0])), full((h, c)), full((1, c)),
        ],
        out_specs=pl.BlockSpec((_R, c), lambda i: (i, 0)),
        out_shape=jax.ShapeDtypeStruct((n, c), jnp.float32),
        compiler_params=seq,
    )(fadj, tf, es, bf2,
      Wa1.astype(lp), ba1.reshape(1, -1), Wa2.reshape(1, -1).astype(lp),
      Wm.astype(lp), bm.reshape(1, -1))
    return out


# speedup vs baseline: 1.0424x; 1.0424x over previous
import jax
import jax.numpy as jnp
from jax.experimental import pallas as pl
from jax.experimental.pallas import tpu as pltpu

_R = 200
_RS = 1000
_H = 128
_LP = jnp.bfloat16


def _pass1_body(x_ref, ws_ref, wf_ref, sadj_ref, fadj_ref, bs1_ref, bf1_ref,
                w12_ref, wc2_ref, w22_ref, ts_ref, tf_ref, ss_ref, sf_ref):
    i = pl.program_id(0)

    @pl.when(i == 0)
    def _():
        xb = x_ref[...].astype(_LP)
        ss_ref[...] = jnp.dot(xb, ws_ref[...],
                              preferred_element_type=jnp.float32).astype(_LP)
        sf_ref[...] = jnp.dot(xb, wf_ref[...],
                              preferred_element_type=jnp.float32).astype(_LP)

    @pl.when(i > 0)
    def _():
        accs = jnp.dot(sadj_ref[...].astype(_LP), ss_ref[...],
                       preferred_element_type=jnp.float32)
        accf = jnp.dot(fadj_ref[...].astype(_LP), sf_ref[...],
                       preferred_element_type=jnp.float32)
        hs = jnp.maximum(accs + bs1_ref[...], 0.0).astype(_LP)
        hf = jnp.maximum(accf + bf1_ref[...], 0.0).astype(_LP)
        ts_ref[:, :_H] = jnp.dot(hs[:, :_H], w12_ref[...],
                                 preferred_element_type=jnp.float32).astype(_LP)
        ts_ref[:, _H:] = jnp.dot(hs[:, _H:], wc2_ref[...],
                                 preferred_element_type=jnp.float32).astype(_LP)
        tf_ref[:, :_H] = jnp.dot(hf[:, :_H], w22_ref[...],
                                 preferred_element_type=jnp.float32).astype(_LP)
        tf_ref[:, _H:] = jnp.dot(hf[:, _H:], wc2_ref[...],
                                 preferred_element_type=jnp.float32).astype(_LP)


def _pass2_body(sadj_ref, fadj_ref, ts_ref, tf_ref, bs2_ref, bf2_ref,
                wa1_ref, ba1_ref, wa2_ref, wm_ref, bm_ref, out_ref):
    es = jnp.dot(sadj_ref[...].astype(_LP), ts_ref[...],
                 preferred_element_type=jnp.float32) + bs2_ref[...]
    ef = jnp.dot(fadj_ref[...].astype(_LP), tf_ref[...],
                 preferred_element_type=jnp.float32) + bf2_ref[...]
    emb1 = es[:, :_H]
    emb2 = ef[:, :_H]
    xcom = 0.5 * (es[:, _H:] + ef[:, _H:])
    wa1 = wa1_ref[...]
    ba1 = ba1_ref[...]
    wa2 = wa2_ref[...]

    def att(e):
        t = jnp.tanh(jnp.dot(e.astype(_LP), wa1,
                             preferred_element_type=jnp.float32) + ba1)
        return jnp.sum(t.astype(_LP).astype(jnp.float32)
                       * wa2.astype(jnp.float32), axis=1, keepdims=True)

    w1 = att(emb1)
    w2 = att(emb2)
    w3 = att(xcom)
    m = jnp.maximum(jnp.maximum(w1, w2), w3)
    e1 = jnp.exp(w1 - m)
    e2 = jnp.exp(w2 - m)
    e3 = jnp.exp(w3 - m)
    emb = (e1 * emb1 + e2 * emb2 + e3 * xcom) / (e1 + e2 + e3)
    logits = jnp.dot(emb.astype(_LP), wm_ref[...],
                     preferred_element_type=jnp.float32) + bm_ref[...]
    lmax = jnp.max(logits, axis=1, keepdims=True)
    lse = jnp.log(jnp.sum(jnp.exp(logits - lmax), axis=1, keepdims=True)) + lmax
    out_ref[...] = logits - lse


def kernel(x, sadj, fadj, W1_1, b1_1, W1_2, b1_2, W2_1, b2_1, W2_2, b2_2,
           Wc_1, bc_1, Wc_2, bc_2, Wa1, ba1, Wa2, Wm, bm):
    n, f = x.shape
    h = W1_1.shape[1]
    c = Wm.shape[1]
    lp = _LP

    ws1 = jnp.concatenate([W1_1, Wc_1], axis=1).astype(lp)
    wf1 = jnp.concatenate([W2_1, Wc_1], axis=1).astype(lp)
    bs1 = jnp.concatenate([b1_1, bc_1]).reshape(1, 2 * h)
    bf1 = jnp.concatenate([b2_1, bc_1]).reshape(1, 2 * h)
    bs2 = jnp.concatenate([b1_2, bc_2]).reshape(1, 2 * h)
    bf2 = jnp.concatenate([b2_2, bc_2]).reshape(1, 2 * h)

    full = lambda shape: pl.BlockSpec(shape, lambda i: (0, 0))
    sup_spec = full((n, 2 * h))
    adj_spec = pl.BlockSpec((_R, n), lambda i: (i, 0))
    row_spec = pl.BlockSpec((_R, 2 * h), lambda i: (i, 0))
    seq = pltpu.CompilerParams(dimension_semantics=("arbitrary",))
    w2c = Wc_2.astype(lp)

    shift_adj = pl.BlockSpec((_R, n), lambda i: (jnp.maximum(i - 1, 0), 0))
    shift_row = pl.BlockSpec((_R, 2 * h), lambda i: (jnp.maximum(i - 1, 0), 0))
    ts, tf = pl.pallas_call(
        _pass1_body,
        grid=(n // _R + 1,),
        in_specs=[full((n, f)), full((f, 2 * h)), full((f, 2 * h)),
                  shift_adj, shift_adj,
                  full((1, 2 * h)), full((1, 2 * h)),
                  full((h, h)), full((h, h)), full((h, h))],
        out_specs=[shift_row, shift_row],
        out_shape=[jax.ShapeDtypeStruct((n, 2 * h), lp)] * 2,
        scratch_shapes=[pltpu.VMEM((n, 2 * h), lp)] * 2,
        compiler_params=seq,
    )(x, ws1, wf1, sadj, fadj, bs1, bf1,
      W1_2.astype(lp), w2c, W2_2.astype(lp))

    out = pl.pallas_call(
        _pass2_body,
        grid=(n // _R,),
        in_specs=[
            adj_spec, adj_spec, sup_spec, sup_spec,
            full((1, 2 * h)), full((1, 2 * h)),
            full((h, Wa1.shape[1])), full((1, Wa1.shape[1])),
            full((1, Wa2.shape[0])), full((h, c)), full((1, c)),
        ],
        out_specs=pl.BlockSpec((_R, c), lambda i: (i, 0)),
        out_shape=jax.ShapeDtypeStruct((n, c), jnp.float32),
        compiler_params=seq,
    )(sadj, fadj, ts, tf, bs2, bf2,
      Wa1.astype(lp), ba1.reshape(1, -1), Wa2.reshape(1, -1).astype(lp),
      Wm.astype(lp), bm.reshape(1, -1))
    return out
